# baseline (speedup 1.0000x reference)
import functools

import jax
import jax.numpy as jnp
from jax import lax
from jax.experimental import pallas as pl
from jax.experimental.pallas import tpu as pltpu
from jax.experimental.pallas import tpu_sc as plsc

VOCAB = 1000000
EMBED_DIM = 64
BATCH = 4096
FIELDS = 26

_B = BATCH * FIELDS
_VP = VOCAB // 2

_info = plsc.get_sparse_core_info()
_NC, _NS = _info.num_cores, _info.num_subcores
_NW = _NC * _NS
_B_PER_W = _B // _NW
_CH = 128
_N_CHUNKS = _B_PER_W // _CH
_L = 16


def _make_kernel():
    mesh = plsc.VectorSubcoreMesh(core_axis_name="c", subcore_axis_name="s")

    @functools.partial(
        pl.kernel,
        mesh=mesh,
        out_type=jax.ShapeDtypeStruct((_B, 2, EMBED_DIM), jnp.float32),
        scratch_types=[
            pltpu.VMEM((_B_PER_W,), jnp.int32),
            pltpu.VMEM((_CH, 2, EMBED_DIM), jnp.float32),
            pltpu.VMEM((_CH, 2, EMBED_DIM), jnp.float32),
            pltpu.SemaphoreType.DMA,
            pltpu.SemaphoreType.DMA,
            pltpu.SemaphoreType.DMA,
            pltpu.SemaphoreType.DMA,
        ],
    )
    def gather_kernel(table_hbm, idx_hbm, out_hbm, idx_v,
                      rows0, rows1, g0, g1, o0, o1):
        wid = lax.axis_index("s") * _NC + lax.axis_index("c")
        base = pl.multiple_of(wid * _B_PER_W, _B_PER_W)
        pltpu.sync_copy(idx_hbm.at[pl.ds(base, _B_PER_W)], idx_v)

        bufs = (rows0, rows1)
        gsems = (g0, g1)
        osems = (o0, o1)

        def fill(c):
            buf = bufs[c % 2]
            sem = gsems[c % 2]

            def body(g, carry):
                v = idx_v[pl.ds(c * _CH + g * _L, _L)]
                for t in range(_L):
                    p = v[t] >> 1
                    pltpu.async_copy(
                        table_hbm.at[pl.ds(p, 1)],
                        buf.at[pl.ds(g * _L + t, 1)],
                        sem,
                    )
                return carry

            lax.fori_loop(0, _CH // _L, body, 0)

        def drain(c):
            pltpu.make_async_copy(
                out_hbm.at[pl.ds(0, _CH)], bufs[c % 2], gsems[c % 2]
            ).wait()

        def put(c):
            return pltpu.async_copy(
                bufs[c % 2], out_hbm.at[pl.ds(base + c * _CH, _CH)],
                osems[c % 2],
            )

        puts = [None, None]
        for c in range(_N_CHUNKS):
            b = c % 2
            if c >= 2:
                puts[b].wait()
            fill(c)
            drain(c)
            puts[b] = put(c)
        puts[(_N_CHUNKS - 2) % 2].wait()
        puts[(_N_CHUNKS - 1) % 2].wait()

    return gather_kernel


_gather = _make_kernel()


def kernel(inp, weight):
    idx = inp.reshape(-1).astype(jnp.int32)
    pairs_table = weight.reshape(_VP, 2, EMBED_DIM)
    pairs = _gather(pairs_table, idx)
    odd = ((idx & 1) == 1)[:, None]
    out_flat = jnp.where(odd, pairs[:, 1, :], pairs[:, 0, :])
    return out_flat.reshape(BATCH, FIELDS, EMBED_DIM)


# speedup vs baseline: 3.7144x; 3.7144x over previous
import functools

import jax
import jax.numpy as jnp
from jax import lax
from jax.experimental import pallas as pl
from jax.experimental.pallas import tpu as pltpu
from jax.experimental.pallas import tpu_sc as plsc

VOCAB = 1000000
EMBED_DIM = 64
BATCH = 4096
FIELDS = 26

_B = BATCH * FIELDS
_VT = VOCAB // 8

_info = plsc.get_sparse_core_info()
_NC, _NS = _info.num_cores, _info.num_subcores
_NW = _NC * _NS
_B_PER_W = _B // _NW
_CH = 208
_NP = _CH // 2
_N_CHUNKS = _B_PER_W // _CH
_L = 16


def _make_kernel():
    mesh = plsc.VectorSubcoreMesh(core_axis_name="c", subcore_axis_name="s")

    @functools.partial(
        pl.kernel,
        mesh=mesh,
        out_type=jax.ShapeDtypeStruct((BATCH, FIELDS, EMBED_DIM),
                                      jnp.float32),
        scratch_types=[
            pltpu.VMEM((_B_PER_W,), jnp.int32),
            pltpu.VMEM((_NP, 2, EMBED_DIM), jnp.float32),
            pltpu.VMEM((_NP, 2, EMBED_DIM), jnp.float32),
            pltpu.SemaphoreType.DMA,
            pltpu.SemaphoreType.DMA,
            pltpu.SemaphoreType.DMA,
            pltpu.SemaphoreType.DMA,
        ],
    )
    def gather_kernel(table_hbm, idx_hbm, out_hbm, idx_v,
                      rows0, rows1, g0, g1, o0, o1):
        wid = lax.axis_index("s") * _NC + lax.axis_index("c")
        base = pl.multiple_of(wid * _B_PER_W, _B_PER_W)
        b_base = pl.multiple_of(wid * (BATCH // _NW), BATCH // _NW)
        pltpu.sync_copy(idx_hbm.at[pl.ds(base, _B_PER_W)], idx_v)

        bufs = (rows0, rows1)
        gsems = (g0, g1)
        osems = (o0, o1)

        def fill(c):
            buf = bufs[c % 2]
            sem = gsems[c % 2]

            def body(g, carry):
                v = idx_v[pl.ds(c * _CH + g * _L, _L)]
                for t in range(_L):
                    i = v[t]
                    pltpu.async_copy(
                        table_hbm.at[pl.ds(i >> 3, 1), pl.ds(i & 7, 1),
                                     pl.ds(0, EMBED_DIM)],
                        buf.at[pl.ds(g * (_L // 2) + t // 2, 1),
                               pl.ds(t & 1, 1), pl.ds(0, EMBED_DIM)],
                        sem,
                    )
                return carry

            lax.fori_loop(0, _CH // _L, body, 0)

        def drain(c, sems):
            pltpu.make_async_copy(
                table_hbm.at[pl.ds(0, _NP), pl.ds(0, 2), pl.ds(0, EMBED_DIM)],
                bufs[c % 2], sems[c % 2]
            ).wait()

        def put_pairs(c):
            buf = bufs[c % 2]
            sem = osems[c % 2]
            row0 = c * _CH

            def body(g, carry):
                for t in range(8):
                    q = g * 8 + t
                    rj = row0 + 2 * q
                    b = b_base + rj // FIELDS
                    f0 = rj % FIELDS
                    pltpu.async_copy(
                        buf.at[pl.ds(q, 1)],
                        out_hbm.at[pl.ds(b, 1), pl.ds(f0, 2),
                                   pl.ds(0, EMBED_DIM)],
                        sem,
                    )
                return carry

            lax.fori_loop(0, _NP // 8, body, 0)

        for c in range(_N_CHUNKS):
            if c >= 2:
                drain(c, osems)
            fill(c)
            drain(c, gsems)
            put_pairs(c)
        drain(_N_CHUNKS - 2, osems)
        drain(_N_CHUNKS - 1, osems)

    return gather_kernel


_gather = _make_kernel()


def kernel(inp, weight):
    idx = inp.reshape(-1).astype(jnp.int32)
    table_tiles = weight.reshape(_VT, 8, EMBED_DIM)
    return _gather(table_tiles, idx)
